# baseline (speedup 1.0000x reference)
import functools

import jax
import jax.numpy as jnp
import numpy as np
from jax.experimental import pallas as pl
from jax.experimental.pallas import tpu as pltpu

_NUM_CLASSES = 80
_C = _NUM_CLASSES + 6
_ANCHORS = [[12.0, 16.0], [19.0, 36.0], [40.0, 28.0]]
_ANGLES = [-1.0471975511965976, -0.5235987755982988, 0.0,
           0.5235987755982988, 1.0471975511965976, 1.5707963267948966]
_STRIDE = 8
_SCALE_XY = 1.05
_MASKED_NP = np.array([(aw / _STRIDE, ah / _STRIDE, a)
                       for aw, ah in _ANCHORS for a in _ANGLES],
                      dtype=np.float32)
_NA = _MASKED_NP.shape[0]


def _decode_body(m_ref, x_ref, o_ref, *, S, G):
    x = x_ref[0, 0]
    top = x[0:8, :]
    rest = x[8:, :]

    rest_y = jnp.tanh(rest * 0.5) * 0.5 + 0.5

    rid = jax.lax.broadcasted_iota(jnp.int32, (8, S), 0)
    sig = jnp.tanh(top * 0.5) * 0.5 + 0.5
    e = jnp.exp(top)

    k = pl.program_id(2)
    col = jax.lax.broadcasted_iota(jnp.int32, (8, S), 1) + k * S
    gx = (col % G).astype(jnp.float32)
    gy = (col // G).astype(jnp.float32)

    aw8 = m_ref[0, 0, 0]
    ah8 = m_ref[0, 0, 1]
    aa = m_ref[0, 0, 2]

    xy = sig * (8.0 * _SCALE_XY) + (8.0 * jnp.where(rid == 0, gx, gy)
                                    - 8.0 * (_SCALE_XY - 1.0) / 2.0)
    wh = e * jnp.where(rid == 2, aw8, ah8)
    top_y = jnp.where(rid < 2, xy,
                      jnp.where(rid < 4, wh,
                                jnp.where(rid == 4, top + aa, sig)))

    y = jnp.concatenate([top_y, rest_y], axis=0)
    o_ref[0, 0] = y.T


def kernel(output):
    B = output.shape[0]
    G = output.shape[2]
    GG = G * G
    S = 1024
    nk = GG // S

    x = output.reshape(B, _NA, _C, GG)
    masked8 = np.zeros((_NA, 1, 8), dtype=np.float32)
    masked8[:, 0, 0] = _MASKED_NP[:, 0] * _STRIDE
    masked8[:, 0, 1] = _MASKED_NP[:, 1] * _STRIDE
    masked8[:, 0, 2] = _MASKED_NP[:, 2]
    m = jnp.asarray(masked8)

    out = pl.pallas_call(
        functools.partial(_decode_body, S=S, G=G),
        grid=(B, _NA, nk),
        in_specs=[
            pl.BlockSpec((1, 1, 8), lambda b, a, k: (a, 0, 0),
                         memory_space=pltpu.SMEM),
            pl.BlockSpec((1, 1, _C, S), lambda b, a, k: (b, a, 0, k)),
        ],
        out_specs=pl.BlockSpec((1, 1, S, _C), lambda b, a, k: (b, a, k, 0)),
        out_shape=jax.ShapeDtypeStruct((B, _NA, GG, _C), jnp.float32),
    )(m, x)
    return out.reshape(B, _NA * GG, _C)


# speedup vs baseline: 1.3695x; 1.3695x over previous
import functools

import jax
import jax.numpy as jnp
import numpy as np
from jax.experimental import pallas as pl
from jax.experimental.pallas import tpu as pltpu

_NUM_CLASSES = 80
_C = _NUM_CLASSES + 6
_ANCHORS = [[12.0, 16.0], [19.0, 36.0], [40.0, 28.0]]
_ANGLES = [-1.0471975511965976, -0.5235987755982988, 0.0,
           0.5235987755982988, 1.0471975511965976, 1.5707963267948966]
_STRIDE = 8
_SCALE_XY = 1.05
_MASKED_NP = np.array([(aw / _STRIDE, ah / _STRIDE, a)
                       for aw, ah in _ANCHORS for a in _ANGLES],
                      dtype=np.float32)
_NA = _MASKED_NP.shape[0]


def _decode_body(m_ref, x_ref, o_ref, *, YC, G):
    x = x_ref[0, 0]
    top = x[0:8]
    rest = x[8:]

    rest_y = jnp.tanh(rest * 0.5) * 0.5 + 0.5

    rid = jax.lax.broadcasted_iota(jnp.int32, (8, YC, G), 0)
    sig = jnp.tanh(top * 0.5) * 0.5 + 0.5
    e = jnp.exp(top)

    k = pl.program_id(2)
    gx = jax.lax.broadcasted_iota(jnp.int32, (8, YC, G), 2).astype(jnp.float32)
    gy = (jax.lax.broadcasted_iota(jnp.int32, (8, YC, G), 1)
          + k * YC).astype(jnp.float32)

    aw8 = m_ref[0, 0, 0]
    ah8 = m_ref[0, 0, 1]
    aa = m_ref[0, 0, 2]

    xy = sig * (8.0 * _SCALE_XY) + (8.0 * jnp.where(rid == 0, gx, gy)
                                    - 8.0 * (_SCALE_XY - 1.0) / 2.0)
    wh = e * jnp.where(rid == 2, aw8, ah8)
    top_y = jnp.where(rid < 2, xy,
                      jnp.where(rid < 4, wh,
                                jnp.where(rid == 4, top + aa, sig)))

    y = jnp.concatenate([top_y, rest_y], axis=0)
    for j in range(YC):
        o_ref[0, 0, j] = y[:, j, :].T


def kernel(output):
    B = output.shape[0]
    G = output.shape[2]
    YC = 16
    nk = G // YC

    x = output.reshape(B, _NA, _C, G, G)
    masked8 = np.zeros((_NA, 1, 8), dtype=np.float32)
    masked8[:, 0, 0] = _MASKED_NP[:, 0] * _STRIDE
    masked8[:, 0, 1] = _MASKED_NP[:, 1] * _STRIDE
    masked8[:, 0, 2] = _MASKED_NP[:, 2]
    m = jnp.asarray(masked8)

    out = pl.pallas_call(
        functools.partial(_decode_body, YC=YC, G=G),
        grid=(B, _NA, nk),
        in_specs=[
            pl.BlockSpec((1, 1, 8), lambda b, a, k: (a, 0, 0),
                         memory_space=pltpu.SMEM),
            pl.BlockSpec((1, 1, _C, YC, G), lambda b, a, k: (b, a, 0, k, 0)),
        ],
        out_specs=pl.BlockSpec((1, 1, YC, G, _C),
                               lambda b, a, k: (b, a, k, 0, 0)),
        out_shape=jax.ShapeDtypeStruct((B, _NA, G, G, _C), jnp.float32),
    )(m, x)
    return out.reshape(B, _NA * G * G, _C)


# speedup vs baseline: 1.8544x; 1.3541x over previous
import functools

import jax
import jax.numpy as jnp
import numpy as np
from jax.experimental import pallas as pl
from jax.experimental.pallas import tpu as pltpu

_NUM_CLASSES = 80
_C = _NUM_CLASSES + 6
_ANCHORS = [[12.0, 16.0], [19.0, 36.0], [40.0, 28.0]]
_ANGLES = [-1.0471975511965976, -0.5235987755982988, 0.0,
           0.5235987755982988, 1.0471975511965976, 1.5707963267948966]
_STRIDE = 8
_SCALE_XY = 1.05
_MASKED_NP = np.array([(aw / _STRIDE, ah / _STRIDE, a)
                       for aw, ah in _ANCHORS for a in _ANGLES],
                      dtype=np.float32)
_NA = _MASKED_NP.shape[0]


def _decode_body(m_ref, x_ref, o_ref, *, YC, G):
    x = x_ref[0, 0]
    top = x[0:8]
    rest = x[8:]

    rest_y = jnp.tanh(rest * 0.5) * 0.5 + 0.5

    rid = jax.lax.broadcasted_iota(jnp.int32, (8, YC, G), 0)
    sig = jnp.tanh(top * 0.5) * 0.5 + 0.5
    e = jnp.exp(top)

    k = pl.program_id(2)
    gx = jax.lax.broadcasted_iota(jnp.int32, (8, YC, G), 2).astype(jnp.float32)
    gy = (jax.lax.broadcasted_iota(jnp.int32, (8, YC, G), 1)
          + k * YC).astype(jnp.float32)

    aw8 = m_ref[0, 0, 0]
    ah8 = m_ref[0, 0, 1]
    aa = m_ref[0, 0, 2]

    xy = sig * (8.0 * _SCALE_XY) + (8.0 * jnp.where(rid == 0, gx, gy)
                                    - 8.0 * (_SCALE_XY - 1.0) / 2.0)
    wh = e * jnp.where(rid == 2, aw8, ah8)
    top_y = jnp.where(rid < 2, xy,
                      jnp.where(rid < 4, wh,
                                jnp.where(rid == 4, top + aa, sig)))

    y = jnp.concatenate([top_y, rest_y], axis=0)
    y2 = y.reshape(_C, YC * G)
    o_ref[0, 0] = y2.T


def kernel(output):
    B = output.shape[0]
    G = output.shape[2]
    YC = 64
    nk = G // YC

    x = output.reshape(B, _NA, _C, G, G)
    masked8 = np.zeros((_NA, 1, 8), dtype=np.float32)
    masked8[:, 0, 0] = _MASKED_NP[:, 0] * _STRIDE
    masked8[:, 0, 1] = _MASKED_NP[:, 1] * _STRIDE
    masked8[:, 0, 2] = _MASKED_NP[:, 2]
    m = jnp.asarray(masked8)

    out = pl.pallas_call(
        functools.partial(_decode_body, YC=YC, G=G),
        grid=(B, _NA, nk),
        in_specs=[
            pl.BlockSpec((1, 1, 8), lambda b, a, k: (a, 0, 0),
                         memory_space=pltpu.SMEM),
            pl.BlockSpec((1, 1, _C, YC, G), lambda b, a, k: (b, a, 0, k, 0)),
        ],
        out_specs=pl.BlockSpec((1, 1, YC * G, _C),
                               lambda b, a, k: (b, a, k, 0)),
        out_shape=jax.ShapeDtypeStruct((B, _NA, G * G, _C), jnp.float32),
    )(m, x)
    return out.reshape(B, _NA * G * G, _C)
